# baseline (speedup 1.0000x reference)
import functools

import jax
import jax.numpy as jnp
from jax import lax
from jax.experimental import pallas as pl
from jax.experimental.pallas import tpu as pltpu
from jax.experimental.pallas import tpu_sc as plsc

_N = 10000
_T = 20000
_E = 320000
_C = 128
_OUT = 10

_NC = 2
_NS = 16
_NW = _NC * _NS

_NP = 10240
_ROWS_SUB = _NP // _NS
_XPW = _NP // _NW
_XCH = 80
_ECH = 128
_EPW = 10240
_EP = _EPW * _NW
_NCH = _EPW // _ECH
_DW = 128
_ZR = 64

_BLK = 1024
_GRID = _NP // _BLK


@functools.cache
def _get_mesh():
    return plsc.VectorSubcoreMesh(core_axis_name="c", subcore_axis_name="s",
                                  num_cores=_NC, num_subcores=_NS)


@functools.cache
def _get_gather_x_deg():
    @functools.partial(
        pl.kernel,
        out_type=[
            jax.ShapeDtypeStruct((_NP, _C), jnp.float32),
            jax.ShapeDtypeStruct((_NC * _NP, _DW), jnp.float32),
        ],
        mesh=_get_mesh(),
        scratch_types=[
            pltpu.VMEM((_XCH,), jnp.int32),
            pltpu.VMEM((_XCH, _C), jnp.float32),
            pltpu.VMEM((_ECH,), jnp.int32),
            pltpu.VMEM((_ECH, _DW), jnp.float32),
            pltpu.VMEM((_ZR, _DW), jnp.float32),
            pltpu.VMEM_SHARED((_NP, _DW), jnp.float32),
            pltpu.SemaphoreType.DMA,
        ],
    )
    def _sc_gather_x_deg(tbl_hbm, nidx_hbm, dst_hbm, x_hbm, deg_hbm,
                         idx_v, rows_v, didx, ones_v, z16, dacc, sem):
        cid = lax.axis_index("c")
        sid = lax.axis_index("s")
        wid = sid * _NC + cid

        z = jnp.zeros((16,), jnp.float32)
        o = jnp.ones((16,), jnp.float32)

        @pl.loop(0, _ECH)
        def _(r):
            for j in range(_DW // 16):
                ones_v[r, pl.ds(j * 16, 16)] = o

        @pl.loop(0, _ZR)
        def _(r):
            for j in range(_DW // 16):
                z16[r, pl.ds(j * 16, 16)] = z

        rbase = sid * _ROWS_SUB
        for j in range(_ROWS_SUB // _ZR):
            pltpu.sync_copy(z16, dacc.at[pl.ds(rbase + j * _ZR, _ZR)])
        plsc.subcore_barrier()

        base = wid * _XPW
        for j in range(_XPW // _XCH):
            off = base + j * _XCH
            pltpu.sync_copy(nidx_hbm.at[pl.ds(off, _XCH)], idx_v)
            pltpu.async_copy(tbl_hbm.at[idx_v], rows_v, sem).wait()
            pltpu.sync_copy(rows_v, x_hbm.at[pl.ds(off, _XCH)])

        ebase = wid * _EPW

        @pl.loop(0, _EPW // _ECH)
        def _(ci):
            off = ebase + ci * _ECH
            pltpu.sync_copy(dst_hbm.at[pl.ds(off, _ECH)], didx)
            pltpu.sync_copy(ones_v, dacc.at[didx], add=True)

        plsc.subcore_barrier()
        obase = cid * _NP + rbase
        for j in range(_ROWS_SUB // _ECH):
            pltpu.sync_copy(dacc.at[pl.ds(rbase + j * _ECH, _ECH)],
                            deg_hbm.at[pl.ds(obase + j * _ECH, _ECH)])

    return _sc_gather_x_deg


@functools.cache
def _get_agg():
    @functools.partial(
        pl.kernel,
        out_type=jax.ShapeDtypeStruct((_NC * _NP, _C), jnp.float32),
        mesh=_get_mesh(),
        scratch_types=[
            pltpu.VMEM((2, _ECH), jnp.int32),
            pltpu.VMEM((2, _ECH), jnp.int32),
            pltpu.VMEM((_ECH, _C), jnp.float32),
            pltpu.VMEM((_ECH, _C), jnp.float32),
            pltpu.VMEM((_ZR, _C), jnp.float32),
            pltpu.VMEM_SHARED((_NP, _C), jnp.float32),
            pltpu.SemaphoreType.DMA,
            pltpu.SemaphoreType.DMA,
            pltpu.SemaphoreType.DMA,
            pltpu.SemaphoreType.DMA,
        ],
    )
    def _sc_agg(eidx_hbm, vals_hbm, acc_hbm,
                e0, e1, r0, r1, zbuf, acc, isem0, isem1, gsem0, gsem1):
        cid = lax.axis_index("c")
        sid = lax.axis_index("s")
        wid = sid * _NC + cid

        z = jnp.zeros((16,), jnp.float32)

        @pl.loop(0, _ZR)
        def _(r):
            for j in range(_C // 16):
                zbuf[r, pl.ds(j * 16, 16)] = z

        rbase = sid * _ROWS_SUB
        for j in range(_ROWS_SUB // _ZR):
            pltpu.sync_copy(zbuf, acc.at[pl.ds(rbase + j * _ZR, _ZR)])
        plsc.subcore_barrier()

        cbase = wid * _NCH
        last = cbase + _NCH - 1

        def load_idx(c, ebuf, isem):
            cr = lax.min(c, last)
            pltpu.async_copy(eidx_hbm.at[cr], ebuf, isem)

        def gather(ebuf, rbuf, gsem):
            pltpu.async_copy(vals_hbm.at[ebuf.at[0]], rbuf, gsem)

        load_idx(cbase, e0, isem0)
        load_idx(cbase + 1, e1, isem1)
        pltpu.make_async_copy(eidx_hbm.at[cbase], e0, isem0).wait()
        gather(e0, r0, gsem0)

        def half(c, ea, ra, isa, gsa, eb, rb, isb, gsb):
            pltpu.make_async_copy(eidx_hbm.at[cbase], eb, isb).wait()
            pltpu.make_async_copy(vals_hbm.at[ea.at[0]], ra, gsa).wait()
            gather(eb, rb, gsb)
            pltpu.sync_copy(ra, acc.at[ea.at[1]], add=True)
            load_idx(c + 2, ea, isa)

        @pl.loop(0, _NCH // 2)
        def _(i):
            c = cbase + 2 * i
            half(c, e0, r0, isem0, gsem0, e1, r1, isem1, gsem1)
            half(c + 1, e1, r1, isem1, gsem1, e0, r0, isem0, gsem0)

        pltpu.make_async_copy(eidx_hbm.at[cbase], e1, isem1).wait()
        pltpu.make_async_copy(vals_hbm.at[e0.at[0]], r0, gsem0).wait()

        plsc.subcore_barrier()
        obase = cid * _NP + rbase
        for j in range(_ROWS_SUB // _ECH):
            pltpu.sync_copy(acc.at[pl.ds(rbase + j * _ECH, _ECH)],
                            acc_hbm.at[pl.ds(obase + j * _ECH, _ECH)])

    return _sc_agg


def _tc1_body(x_ref, p0_ref, p1_ref, d0_ref, d1_ref, wr_ref, wn_ref, b_ref,
              h_ref):
    deg = jnp.maximum(d0_ref[:, :1] + d1_ref[:, :1], 1.0)
    agg = (p0_ref[...] + p1_ref[...]) / deg
    h = (jnp.dot(x_ref[...], wr_ref[...], preferred_element_type=jnp.float32)
         + jnp.dot(agg, wn_ref[...], preferred_element_type=jnp.float32)
         + b_ref[...])
    h_ref[...] = jnp.maximum(h, 0.0)


def _tc2_body(h_ref, p0_ref, p1_ref, d0_ref, d1_ref, wr_ref, wn_ref, b_ref,
              wh_ref, bh_ref, out_ref):
    deg = jnp.maximum(d0_ref[:, :1] + d1_ref[:, :1], 1.0)
    agg = (p0_ref[...] + p1_ref[...]) / deg
    h2 = (jnp.dot(h_ref[...], wr_ref[...], preferred_element_type=jnp.float32)
          + jnp.dot(agg, wn_ref[...], preferred_element_type=jnp.float32)
          + b_ref[...])
    out_ref[...] = (jnp.dot(h2, wh_ref[...],
                            preferred_element_type=jnp.float32) + bh_ref[...])


def _row_specs():
    return [
        pl.BlockSpec((_BLK, _C), lambda i: (i, 0)),
        pl.BlockSpec((_BLK, _C), lambda i: (i, 0)),
        pl.BlockSpec((_BLK, _C), lambda i: (i + _GRID, 0)),
        pl.BlockSpec((_BLK, _DW), lambda i: (i, 0)),
        pl.BlockSpec((_BLK, _DW), lambda i: (i + _GRID, 0)),
        pl.BlockSpec((_C, _C), lambda i: (0, 0)),
        pl.BlockSpec((_C, _C), lambda i: (0, 0)),
        pl.BlockSpec((1, _C), lambda i: (0, 0)),
    ]


_tc1 = pl.pallas_call(
    _tc1_body,
    out_shape=jax.ShapeDtypeStruct((_NP, _C), jnp.float32),
    grid=(_GRID,),
    in_specs=_row_specs(),
    out_specs=pl.BlockSpec((_BLK, _C), lambda i: (i, 0)),
)

_tc2 = pl.pallas_call(
    _tc2_body,
    out_shape=jax.ShapeDtypeStruct((_NP, _OUT), jnp.float32),
    grid=(_GRID,),
    in_specs=_row_specs() + [
        pl.BlockSpec((_C, _OUT), lambda i: (0, 0)),
        pl.BlockSpec((1, _OUT), lambda i: (0, 0)),
    ],
    out_specs=pl.BlockSpec((_BLK, _OUT), lambda i: (i, 0)),
)


def kernel(feat_table, node_idx, edge_index,
           W_root1, W_nbr1, b1, W_root2, W_nbr2, b2, W_head, b_head):
    nidx = jnp.concatenate(
        [node_idx, jnp.zeros((_NP - _N,), jnp.int32)])
    src = jnp.concatenate(
        [edge_index[0], jnp.zeros((_EP - _E,), jnp.int32)])
    dst = jnp.concatenate(
        [edge_index[1], jnp.full((_EP - _E,), _N, jnp.int32)])

    eidx = jnp.stack([src.reshape(-1, _ECH), dst.reshape(-1, _ECH)], axis=1)

    x, deg = _get_gather_x_deg()(feat_table, nidx, dst)
    acc1 = _get_agg()(eidx, x)
    h = _tc1(x, acc1, acc1, deg, deg, W_root1, W_nbr1, b1.reshape(1, _C))
    acc2 = _get_agg()(eidx, h)
    out = _tc2(h, acc2, acc2, deg, deg, W_root2, W_nbr2, b2.reshape(1, _C),
               W_head, b_head.reshape(1, _OUT))
    return out[:_N]


# speedup vs baseline: 2.7181x; 2.7181x over previous
import functools

import jax
import jax.numpy as jnp
from jax import lax
from jax.experimental import pallas as pl
from jax.experimental.pallas import tpu as pltpu
from jax.experimental.pallas import tpu_sc as plsc

_N = 10000
_T = 20000
_E = 320000
_C = 128
_OUT = 10

_NC = 2
_NS = 16
_NW = _NC * _NS

_NP = 10240
_ROWS_SUB = _NP // _NS
_XPW = _NP // _NW
_XCH = 80
_ECH = 128
_EPW = 10240
_EP = _EPW * _NW
_NCH = _EPW // _ECH
_DW = 128
_ZR = 64

_BLK = 1024
_GRID = _NP // _BLK


@functools.cache
def _get_mesh():
    return plsc.VectorSubcoreMesh(core_axis_name="c", subcore_axis_name="s",
                                  num_cores=_NC, num_subcores=_NS)


@functools.cache
def _get_gather_x_deg():
    @functools.partial(
        pl.kernel,
        out_type=[
            jax.ShapeDtypeStruct((_NP, _C), jnp.float32),
            jax.ShapeDtypeStruct((_NC * _NP, _DW), jnp.float32),
        ],
        mesh=_get_mesh(),
        scratch_types=[
            pltpu.VMEM((_XCH,), jnp.int32),
            pltpu.VMEM((_XCH, _C), jnp.float32),
            pltpu.VMEM((_ECH,), jnp.int32),
            pltpu.VMEM((_ECH, _DW), jnp.float32),
            pltpu.VMEM((_ZR, _DW), jnp.float32),
            pltpu.VMEM_SHARED((_NP, _DW), jnp.float32),
            pltpu.SemaphoreType.DMA,
        ],
    )
    def _sc_gather_x_deg(tbl_hbm, nidx_hbm, dst_hbm, x_hbm, deg_hbm,
                         idx_v, rows_v, didx, ones_v, z16, dacc, sem):
        cid = lax.axis_index("c")
        sid = lax.axis_index("s")
        wid = sid * _NC + cid

        z = jnp.zeros((16,), jnp.float32)
        o = jnp.ones((16,), jnp.float32)

        @pl.loop(0, _ECH)
        def _(r):
            for j in range(_DW // 16):
                ones_v[r, pl.ds(j * 16, 16)] = o

        @pl.loop(0, _ZR)
        def _(r):
            for j in range(_DW // 16):
                z16[r, pl.ds(j * 16, 16)] = z

        rbase = sid * _ROWS_SUB
        for j in range(_ROWS_SUB // _ZR):
            pltpu.sync_copy(z16, dacc.at[pl.ds(rbase + j * _ZR, _ZR)])
        plsc.subcore_barrier()

        base = wid * _XPW
        for j in range(_XPW // _XCH):
            off = base + j * _XCH
            pltpu.sync_copy(nidx_hbm.at[pl.ds(off, _XCH)], idx_v)
            pltpu.async_copy(tbl_hbm.at[idx_v], rows_v, sem).wait()
            pltpu.sync_copy(rows_v, x_hbm.at[pl.ds(off, _XCH)])

        ebase = wid * _EPW

        @pl.loop(0, _EPW // _ECH)
        def _(ci):
            off = ebase + ci * _ECH
            pltpu.sync_copy(dst_hbm.at[pl.ds(off, _ECH)], didx)
            pltpu.sync_copy(ones_v, dacc.at[didx], add=True)

        plsc.subcore_barrier()
        obase = cid * _NP + rbase
        for j in range(_ROWS_SUB // _ECH):
            pltpu.sync_copy(dacc.at[pl.ds(rbase + j * _ECH, _ECH)],
                            deg_hbm.at[pl.ds(obase + j * _ECH, _ECH)])

    return _sc_gather_x_deg


@functools.cache
def _get_agg():
    @functools.partial(
        pl.kernel,
        out_type=jax.ShapeDtypeStruct((_NC * _NP, _C), jnp.float32),
        mesh=_get_mesh(),
        scratch_types=[
            pltpu.VMEM((2, _ECH), jnp.int32),
            pltpu.VMEM((2, _ECH), jnp.int32),
            pltpu.VMEM((_ECH, _C), jnp.float32),
            pltpu.VMEM((_ECH, _C), jnp.float32),
            pltpu.VMEM((_ZR, _C), jnp.float32),
            pltpu.VMEM_SHARED((_NP, _C), jnp.float32),
            pltpu.SemaphoreType.DMA,
            pltpu.SemaphoreType.DMA,
            pltpu.SemaphoreType.DMA,
            pltpu.SemaphoreType.DMA,
        ],
    )
    def _sc_agg(eidx_hbm, vals_hbm, acc_hbm,
                e0, e1, r0, r1, zbuf, acc, isem0, isem1, gsem0, gsem1):
        cid = lax.axis_index("c")
        sid = lax.axis_index("s")
        wid = sid * _NC + cid

        z = jnp.zeros((16,), jnp.float32)

        @pl.loop(0, _ZR)
        def _(r):
            for j in range(_C // 16):
                zbuf[r, pl.ds(j * 16, 16)] = z

        rbase = sid * _ROWS_SUB
        for j in range(_ROWS_SUB // _ZR):
            pltpu.sync_copy(zbuf, acc.at[pl.ds(rbase + j * _ZR, _ZR)])
        plsc.subcore_barrier()

        cbase = wid * _NCH
        last = cbase + _NCH - 1

        def load_idx(c, ebuf, isem):
            cr = lax.min(c, last)
            pltpu.async_copy(eidx_hbm.at[cr], ebuf, isem)

        def gather(ebuf, rbuf, gsem):
            pltpu.async_copy(vals_hbm.at[ebuf.at[0]], rbuf, gsem)

        load_idx(cbase, e0, isem0)
        load_idx(cbase + 1, e1, isem1)
        pltpu.make_async_copy(eidx_hbm.at[cbase], e0, isem0).wait()
        gather(e0, r0, gsem0)

        def half(c, ea, ra, isa, gsa, eb, rb, isb, gsb):
            pltpu.make_async_copy(eidx_hbm.at[cbase], eb, isb).wait()
            pltpu.make_async_copy(vals_hbm.at[ea.at[0]], ra, gsa).wait()
            gather(eb, rb, gsb)
            pltpu.sync_copy(ra, acc.at[ea.at[1]], add=True)
            load_idx(c + 2, ea, isa)

        @pl.loop(0, _NCH // 2)
        def _(i):
            c = cbase + 2 * i
            half(c, e0, r0, isem0, gsem0, e1, r1, isem1, gsem1)
            half(c + 1, e1, r1, isem1, gsem1, e0, r0, isem0, gsem0)

        pltpu.make_async_copy(eidx_hbm.at[cbase], e1, isem1).wait()
        pltpu.make_async_copy(vals_hbm.at[e0.at[0]], r0, gsem0).wait()

        plsc.subcore_barrier()
        obase = cid * _NP + rbase
        for j in range(_ROWS_SUB // _ECH):
            pltpu.sync_copy(acc.at[pl.ds(rbase + j * _ECH, _ECH)],
                            acc_hbm.at[pl.ds(obase + j * _ECH, _ECH)])

    return _sc_agg


def _tc1_body(x_ref, p0_ref, p1_ref, d0_ref, d1_ref, wr_ref, wn_ref, b_ref,
              h_ref):
    deg = jnp.maximum(d0_ref[:, :1] + d1_ref[:, :1], 1.0)
    agg = (p0_ref[...] + p1_ref[...]) / deg
    h = (jnp.dot(x_ref[...], wr_ref[...], preferred_element_type=jnp.float32)
         + jnp.dot(agg, wn_ref[...], preferred_element_type=jnp.float32)
         + b_ref[...])
    h_ref[...] = jnp.maximum(h, 0.0)


def _tc2_body(h_ref, p0_ref, p1_ref, d0_ref, d1_ref, wr_ref, wn_ref, b_ref,
              wh_ref, bh_ref, out_ref):
    deg = jnp.maximum(d0_ref[:, :1] + d1_ref[:, :1], 1.0)
    agg = (p0_ref[...] + p1_ref[...]) / deg
    h2 = (jnp.dot(h_ref[...], wr_ref[...], preferred_element_type=jnp.float32)
          + jnp.dot(agg, wn_ref[...], preferred_element_type=jnp.float32)
          + b_ref[...])
    out_ref[...] = (jnp.dot(h2, wh_ref[...],
                            preferred_element_type=jnp.float32) + bh_ref[...])


def _row_specs():
    return [
        pl.BlockSpec((_BLK, _C), lambda i: (i, 0)),
        pl.BlockSpec((_BLK, _C), lambda i: (i, 0)),
        pl.BlockSpec((_BLK, _C), lambda i: (i + _GRID, 0)),
        pl.BlockSpec((_BLK, _DW), lambda i: (i, 0)),
        pl.BlockSpec((_BLK, _DW), lambda i: (i + _GRID, 0)),
        pl.BlockSpec((_C, _C), lambda i: (0, 0)),
        pl.BlockSpec((_C, _C), lambda i: (0, 0)),
        pl.BlockSpec((1, _C), lambda i: (0, 0)),
    ]


_tc1 = pl.pallas_call(
    _tc1_body,
    out_shape=jax.ShapeDtypeStruct((_NP, _C), jnp.float32),
    grid=(_GRID,),
    in_specs=_row_specs(),
    out_specs=pl.BlockSpec((_BLK, _C), lambda i: (i, 0)),
)

_tc2 = pl.pallas_call(
    _tc2_body,
    out_shape=jax.ShapeDtypeStruct((_NP, _OUT), jnp.float32),
    grid=(_GRID,),
    in_specs=_row_specs() + [
        pl.BlockSpec((_C, _OUT), lambda i: (0, 0)),
        pl.BlockSpec((1, _OUT), lambda i: (0, 0)),
    ],
    out_specs=pl.BlockSpec((_BLK, _OUT), lambda i: (i, 0)),
)


def kernel(feat_table, node_idx, edge_index,
           W_root1, W_nbr1, b1, W_root2, W_nbr2, b2, W_head, b_head):
    pe = jnp.arange(_EP - _E, dtype=jnp.int32)
    nidx = jnp.concatenate(
        [node_idx, jnp.arange(_NP - _N, dtype=jnp.int32) % _T])
    src = jnp.concatenate([edge_index[0], pe % _N])
    dst = jnp.concatenate([edge_index[1], _N + pe % (_NP - _N)])

    eidx = jnp.stack([src.reshape(-1, _ECH), dst.reshape(-1, _ECH)], axis=1)

    x, deg = _get_gather_x_deg()(feat_table, nidx, dst)
    acc1 = _get_agg()(eidx, x)
    h = _tc1(x, acc1, acc1, deg, deg, W_root1, W_nbr1, b1.reshape(1, _C))
    acc2 = _get_agg()(eidx, h)
    out = _tc2(h, acc2, acc2, deg, deg, W_root2, W_nbr2, b2.reshape(1, _C),
               W_head, b_head.reshape(1, _OUT))
    return out[:_N]


# speedup vs baseline: 3.2452x; 1.1939x over previous
import dataclasses
import functools

import jax
import jax.numpy as jnp
from jax import lax
from jax.experimental import pallas as pl
from jax.experimental.pallas import tpu as pltpu
from jax.experimental.pallas import tpu_sc as plsc

_N = 10000
_T = 20000
_E = 320000
_C = 128
_OUT = 10

_NC = 2
_NS = 16
_NW = _NC * _NS

_NP = 10240
_ROWS_SUB = _NP // _NS
_XPW = _NP // _NW
_XCH = 80
_ECH = 128
_EPW = 10240
_EP = _EPW * _NW
_NCH = _EPW // _ECH
_DW = 128
_ZR = 64

_BLK = 1024
_GRID = _NP // _BLK


@functools.cache
def _get_mesh():
    return plsc.VectorSubcoreMesh(core_axis_name="c", subcore_axis_name="s",
                                  num_cores=_NC, num_subcores=_NS)


@functools.cache
def _get_gather_x_deg():
    @functools.partial(
        pl.kernel,
        out_type=[
            jax.ShapeDtypeStruct((_NP, _C), jnp.float32),
            jax.ShapeDtypeStruct((_NC * _NP,), jnp.float32),
        ],
        mesh=_get_mesh(),
        scratch_types=[
            pltpu.VMEM((_XCH,), jnp.int32),
            pltpu.VMEM((_XCH, _C), jnp.float32),
            pltpu.VMEM((_ECH,), jnp.int32),
            pltpu.VMEM((_ECH,), jnp.int32),
            pltpu.VMEM((_NP,), jnp.float32),
            pltpu.VMEM((_ROWS_SUB,), jnp.float32),
            pltpu.VMEM((_ROWS_SUB,), jnp.float32),
            pltpu.VMEM_SHARED((_NS * _NP,), jnp.float32),
            pltpu.SemaphoreType.DMA,
            pltpu.SemaphoreType.DMA,
            pltpu.SemaphoreType.DMA,
        ],
        compiler_params=dataclasses.replace(pltpu.CompilerParams(),
                                            needs_layout_passes=False),
    )
    def _sc_gather_x_deg(tbl_hbm, nidx_hbm, dst_hbm, x_hbm, deg_hbm,
                         idx_v, rows_v, d0, d1, hbuf, rbuf, abuf, hstage,
                         sem, isem0, isem1):
        cid = lax.axis_index("c")
        sid = lax.axis_index("s")
        wid = sid * _NC + cid

        z = jnp.zeros((16,), jnp.float32)
        o = jnp.ones((16,), jnp.float32)

        @pl.loop(0, _NP // 16)
        def _(i):
            hbuf[pl.ds(i * 16, 16)] = z

        base = wid * _XPW
        for j in range(_XPW // _XCH):
            off = base + j * _XCH
            pltpu.sync_copy(nidx_hbm.at[pl.ds(off, _XCH)], idx_v)
            pltpu.async_copy(tbl_hbm.at[idx_v], rows_v, sem).wait()
            pltpu.sync_copy(rows_v, x_hbm.at[pl.ds(off, _XCH)])

        ebase = wid * _EPW
        elast = ebase + _EPW - _ECH

        def load_dst(off, buf, isem):
            pltpu.async_copy(
                dst_hbm.at[pl.ds(lax.min(off, elast), _ECH)], buf, isem)

        load_dst(ebase, d0, isem0)
        load_dst(ebase + _ECH, d1, isem1)

        def half(off, da, isa, db, isb):
            pltpu.make_async_copy(dst_hbm.at[pl.ds(ebase, _ECH)], da,
                                  isa).wait()
            for j in range(_ECH // 16):
                plsc.addupdate_scatter(hbuf, [da[pl.ds(j * 16, 16)]], o)
            load_dst(off + 2 * _ECH, da, isa)

        @pl.loop(0, _NCH // 2)
        def _(i):
            off = ebase + 2 * i * _ECH
            half(off, d0, isem0, d1, isem1)
            half(off + _ECH, d1, isem1, d0, isem0)

        pltpu.make_async_copy(dst_hbm.at[pl.ds(ebase, _ECH)], d0, isem0).wait()
        pltpu.make_async_copy(dst_hbm.at[pl.ds(ebase, _ECH)], d1, isem1).wait()

        pltpu.sync_copy(hbuf, hstage.at[pl.ds(sid * _NP, _NP)])
        plsc.subcore_barrier()

        rbase = sid * _ROWS_SUB

        @pl.loop(0, _ROWS_SUB // 16)
        def _(i):
            abuf[pl.ds(i * 16, 16)] = z

        @pl.loop(0, _NS)
        def _(k):
            pltpu.sync_copy(hstage.at[pl.ds(k * _NP + rbase, _ROWS_SUB)],
                            rbuf)
            for t in range(_ROWS_SUB // 16):
                sl = pl.ds(t * 16, 16)
                abuf[sl] = abuf[sl] + rbuf[sl]

        pltpu.sync_copy(abuf, deg_hbm.at[pl.ds(cid * _NP + rbase, _ROWS_SUB)])

    return _sc_gather_x_deg


@functools.cache
def _get_agg():
    @functools.partial(
        pl.kernel,
        out_type=jax.ShapeDtypeStruct((_NC * _NP, _C), jnp.float32),
        mesh=_get_mesh(),
        scratch_types=[
            pltpu.VMEM((2, _ECH), jnp.int32),
            pltpu.VMEM((2, _ECH), jnp.int32),
            pltpu.VMEM((_ECH, _C), jnp.float32),
            pltpu.VMEM((_ECH, _C), jnp.float32),
            pltpu.VMEM((_ZR, _C), jnp.float32),
            pltpu.VMEM_SHARED((_NP, _C), jnp.float32),
            pltpu.SemaphoreType.DMA,
            pltpu.SemaphoreType.DMA,
            pltpu.SemaphoreType.DMA,
            pltpu.SemaphoreType.DMA,
        ],
    )
    def _sc_agg(eidx_hbm, vals_hbm, acc_hbm,
                e0, e1, r0, r1, zbuf, acc, isem0, isem1, gsem0, gsem1):
        cid = lax.axis_index("c")
        sid = lax.axis_index("s")
        wid = sid * _NC + cid

        z = jnp.zeros((16,), jnp.float32)

        @pl.loop(0, _ZR)
        def _(r):
            for j in range(_C // 16):
                zbuf[r, pl.ds(j * 16, 16)] = z

        rbase = sid * _ROWS_SUB
        for j in range(_ROWS_SUB // _ZR):
            pltpu.sync_copy(zbuf, acc.at[pl.ds(rbase + j * _ZR, _ZR)])
        plsc.subcore_barrier()

        cbase = wid * _NCH
        last = cbase + _NCH - 1

        def load_idx(c, ebuf, isem):
            cr = lax.min(c, last)
            pltpu.async_copy(eidx_hbm.at[cr], ebuf, isem)

        def gather(ebuf, rbuf, gsem):
            pltpu.async_copy(vals_hbm.at[ebuf.at[0]], rbuf, gsem)

        load_idx(cbase, e0, isem0)
        load_idx(cbase + 1, e1, isem1)
        pltpu.make_async_copy(eidx_hbm.at[cbase], e0, isem0).wait()
        gather(e0, r0, gsem0)

        def half(c, ea, ra, isa, gsa, eb, rb, isb, gsb):
            pltpu.make_async_copy(eidx_hbm.at[cbase], eb, isb).wait()
            pltpu.make_async_copy(vals_hbm.at[ea.at[0]], ra, gsa).wait()
            gather(eb, rb, gsb)
            pltpu.sync_copy(ra, acc.at[ea.at[1]], add=True)
            load_idx(c + 2, ea, isa)

        @pl.loop(0, _NCH // 2)
        def _(i):
            c = cbase + 2 * i
            half(c, e0, r0, isem0, gsem0, e1, r1, isem1, gsem1)
            half(c + 1, e1, r1, isem1, gsem1, e0, r0, isem0, gsem0)

        pltpu.make_async_copy(eidx_hbm.at[cbase], e1, isem1).wait()
        pltpu.make_async_copy(vals_hbm.at[e0.at[0]], r0, gsem0).wait()

        plsc.subcore_barrier()
        obase = cid * _NP + rbase
        for j in range(_ROWS_SUB // _ECH):
            pltpu.sync_copy(acc.at[pl.ds(rbase + j * _ECH, _ECH)],
                            acc_hbm.at[pl.ds(obase + j * _ECH, _ECH)])

    return _sc_agg


def _tc1_body(x_ref, p0_ref, p1_ref, d0_ref, d1_ref, wr_ref, wn_ref, b_ref,
              h_ref):
    deg = jnp.maximum(d0_ref[...] + d1_ref[...], 1.0)
    agg = (p0_ref[...] + p1_ref[...]) / deg
    h = (jnp.dot(x_ref[...], wr_ref[...], preferred_element_type=jnp.float32)
         + jnp.dot(agg, wn_ref[...], preferred_element_type=jnp.float32)
         + b_ref[...])
    h_ref[...] = jnp.maximum(h, 0.0)


def _tc2_body(h_ref, p0_ref, p1_ref, d0_ref, d1_ref, wr_ref, wn_ref, b_ref,
              wh_ref, bh_ref, out_ref):
    deg = jnp.maximum(d0_ref[...] + d1_ref[...], 1.0)
    agg = (p0_ref[...] + p1_ref[...]) / deg
    h2 = (jnp.dot(h_ref[...], wr_ref[...], preferred_element_type=jnp.float32)
          + jnp.dot(agg, wn_ref[...], preferred_element_type=jnp.float32)
          + b_ref[...])
    out_ref[...] = (jnp.dot(h2, wh_ref[...],
                            preferred_element_type=jnp.float32) + bh_ref[...])


def _row_specs():
    return [
        pl.BlockSpec((_BLK, _C), lambda i: (i, 0)),
        pl.BlockSpec((_BLK, _C), lambda i: (i, 0)),
        pl.BlockSpec((_BLK, _C), lambda i: (i + _GRID, 0)),
        pl.BlockSpec((_BLK, 1), lambda i: (i, 0)),
        pl.BlockSpec((_BLK, 1), lambda i: (i + _GRID, 0)),
        pl.BlockSpec((_C, _C), lambda i: (0, 0)),
        pl.BlockSpec((_C, _C), lambda i: (0, 0)),
        pl.BlockSpec((1, _C), lambda i: (0, 0)),
    ]


_tc1 = pl.pallas_call(
    _tc1_body,
    out_shape=jax.ShapeDtypeStruct((_NP, _C), jnp.float32),
    grid=(_GRID,),
    in_specs=_row_specs(),
    out_specs=pl.BlockSpec((_BLK, _C), lambda i: (i, 0)),
)

_tc2 = pl.pallas_call(
    _tc2_body,
    out_shape=jax.ShapeDtypeStruct((_NP, _OUT), jnp.float32),
    grid=(_GRID,),
    in_specs=_row_specs() + [
        pl.BlockSpec((_C, _OUT), lambda i: (0, 0)),
        pl.BlockSpec((1, _OUT), lambda i: (0, 0)),
    ],
    out_specs=pl.BlockSpec((_BLK, _OUT), lambda i: (i, 0)),
)


def kernel(feat_table, node_idx, edge_index,
           W_root1, W_nbr1, b1, W_root2, W_nbr2, b2, W_head, b_head):
    pe = jnp.arange(_EP - _E, dtype=jnp.int32)
    nidx = jnp.concatenate(
        [node_idx, jnp.arange(_NP - _N, dtype=jnp.int32) % _T])
    src = jnp.concatenate([edge_index[0], pe % _N])
    dst = jnp.concatenate([edge_index[1], _N + pe % (_NP - _N)])

    eidx = jnp.stack([src.reshape(-1, _ECH), dst.reshape(-1, _ECH)], axis=1)

    x, deg = _get_gather_x_deg()(feat_table, nidx, dst)
    deg = deg.reshape(_NC * _NP, 1)
    acc1 = _get_agg()(eidx, x)
    h = _tc1(x, acc1, acc1, deg, deg, W_root1, W_nbr1, b1.reshape(1, _C))
    acc2 = _get_agg()(eidx, h)
    out = _tc2(h, acc2, acc2, deg, deg, W_root2, W_nbr2, b2.reshape(1, _C),
               W_head, b_head.reshape(1, _OUT))
    return out[:_N]
